# baseline (speedup 1.0000x reference)
import jax
import jax.numpy as jnp
from jax import lax
from jax.experimental import pallas as pl
from jax.experimental.pallas import tpu as pltpu
from jax.experimental.pallas import tpu_sc as plsc

_N = 577
_ROWS = 55392
_RCHUNK = 96
_NCHUNKS = _ROWS // _RCHUNK
_NW = 32
_KMAX = -(-_NCHUNKS // _NW)
_CPR = 36

_RENORM_HEADS = (2, 3, 5, 7, 8)
_COPY_HEAD = 10


def _tail_idx(r):
    return [jnp.full((16,), r, jnp.int32), jnp.full((16,), _N - 1, jnp.int32)]


def _row_renorm(buf, r, lane0):
    vs = []
    acc = jnp.zeros((16,), jnp.float32)
    for j in range(_CPR):
        v = jnp.maximum(buf[r, pl.ds(j * 16, 16)], 0.0)
        vs.append(v)
        acc = acc + v
    idx = _tail_idx(r)
    t = jnp.maximum(plsc.load_gather(buf, idx), 0.0)
    acc = acc + jnp.where(lane0, t, 0.0)
    s_vec = jnp.full((16,), jnp.sum(acc) + 1e-5, jnp.float32)
    inv = jnp.ones((16,), jnp.float32) / s_vec
    for j in range(_CPR):
        buf[r, pl.ds(j * 16, 16)] = vs[j] * inv
    plsc.store_scatter(buf, idx, t * inv, mask=lane0)


def _row_uniform(buf, r, lane0):
    scale = jnp.float32(1.0 / _N)
    for j in range(_CPR):
        buf[r, pl.ds(j * 16, 16)] = buf[r, pl.ds(j * 16, 16)] * scale
    idx = _tail_idx(r)
    t = plsc.load_gather(buf, idx)
    plsc.store_scatter(buf, idx, t * scale, mask=lane0)


def _sc_body(x_hbm, o_hbm, buf):
    w = lax.axis_index("s") * 2 + lax.axis_index("c")
    lane0 = lax.iota(jnp.int32, 16) < 1

    def round_iter(k, carry):
        c = w + k * _NW

        @pl.when(c < _NCHUNKS)
        def _():
            base = c * _RCHUNK
            pltpu.sync_copy(x_hbm.at[pl.ds(base, _RCHUNK), :], buf)

            def row(r, cy):
                head = ((base + r) // _N) % 12
                is_renorm = head == _RENORM_HEADS[0]
                for hh in _RENORM_HEADS[1:]:
                    is_renorm = jnp.logical_or(is_renorm, head == hh)
                is_uniform = jnp.logical_and(
                    jnp.logical_not(is_renorm), head != _COPY_HEAD
                )

                @pl.when(is_renorm)
                def _():
                    _row_renorm(buf, r, lane0)

                @pl.when(is_uniform)
                def _():
                    _row_uniform(buf, r, lane0)

                return cy

            lax.fori_loop(0, _RCHUNK, row, 0)
            pltpu.sync_copy(buf, o_hbm.at[pl.ds(base, _RCHUNK), :])

        return carry

    lax.fori_loop(0, _KMAX, round_iter, 0)


def kernel(attn_weights):
    b, nh, n, _ = attn_weights.shape
    x2 = attn_weights.reshape(b * nh * n, n)
    mesh = plsc.VectorSubcoreMesh(
        core_axis_name="c", subcore_axis_name="s", num_cores=2, num_subcores=16
    )
    out = pl.kernel(
        _sc_body,
        out_type=jax.ShapeDtypeStruct((b * nh * n, n), jnp.float32),
        mesh=mesh,
        scratch_types=[pltpu.VMEM((_RCHUNK, n), jnp.float32)],
        compiler_params=pltpu.CompilerParams(needs_layout_passes=False),
    )(x2)
    return out.reshape(b, nh, n, n)


# speedup vs baseline: 1.0026x; 1.0026x over previous
import jax
import jax.numpy as jnp
from jax import lax
from jax.experimental import pallas as pl
from jax.experimental.pallas import tpu as pltpu
from jax.experimental.pallas import tpu_sc as plsc

_N = 577
_ROWS = 55392
_RCHUNK = 96
_NCHUNKS = _ROWS // _RCHUNK
_NW = 32
_KMAX = -(-_NCHUNKS // _NW)
_CPR = 36

_RENORM_HEADS = (2, 3, 5, 7, 8)
_COPY_HEAD = 10


def _tail_idx(r):
    return [jnp.full((16,), r, jnp.int32), jnp.full((16,), _N - 1, jnp.int32)]


def _row_renorm(buf, r, lane0):
    vs = []
    acc = jnp.zeros((16,), jnp.float32)
    for j in range(_CPR):
        v = jnp.maximum(buf[r, pl.ds(j * 16, 16)], 0.0)
        vs.append(v)
        acc = acc + v
    idx = _tail_idx(r)
    t = jnp.maximum(plsc.load_gather(buf, idx), 0.0)
    acc = acc + jnp.where(lane0, t, 0.0)
    s_vec = jnp.full((16,), jnp.sum(acc) + 1e-5, jnp.float32)
    inv = jnp.ones((16,), jnp.float32) / s_vec
    for j in range(_CPR):
        buf[r, pl.ds(j * 16, 16)] = vs[j] * inv
    plsc.store_scatter(buf, idx, t * inv, mask=lane0)


def _row_uniform(buf, r, lane0):
    scale = jnp.float32(1.0 / _N)
    for j in range(_CPR):
        buf[r, pl.ds(j * 16, 16)] = buf[r, pl.ds(j * 16, 16)] * scale
    idx = _tail_idx(r)
    t = plsc.load_gather(buf, idx)
    plsc.store_scatter(buf, idx, t * scale, mask=lane0)


def _sc_body(x_hbm, o_hbm, buf):
    w = lax.axis_index("s") * 2 + lax.axis_index("c")
    lane0 = lax.iota(jnp.int32, 16) < 1

    def round_iter(k, carry):
        c = w + k * _NW

        @pl.when(c < _NCHUNKS)
        def _():
            base = c * _RCHUNK
            pltpu.sync_copy(x_hbm.at[pl.ds(base, _RCHUNK), :], buf)

            @plsc.parallel_loop(0, _RCHUNK, unroll=2)
            def row(r):
                head = ((base + r) // _N) % 12
                is_renorm = head == _RENORM_HEADS[0]
                for hh in _RENORM_HEADS[1:]:
                    is_renorm = jnp.logical_or(is_renorm, head == hh)
                is_uniform = jnp.logical_and(
                    jnp.logical_not(is_renorm), head != _COPY_HEAD
                )

                @pl.when(is_renorm)
                def _():
                    _row_renorm(buf, r, lane0)

                @pl.when(is_uniform)
                def _():
                    _row_uniform(buf, r, lane0)
            pltpu.sync_copy(buf, o_hbm.at[pl.ds(base, _RCHUNK), :])

        return carry

    lax.fori_loop(0, _KMAX, round_iter, 0)


def kernel(attn_weights):
    b, nh, n, _ = attn_weights.shape
    x2 = attn_weights.reshape(b * nh * n, n)
    mesh = plsc.VectorSubcoreMesh(
        core_axis_name="c", subcore_axis_name="s", num_cores=2, num_subcores=16
    )
    out = pl.kernel(
        _sc_body,
        out_type=jax.ShapeDtypeStruct((b * nh * n, n), jnp.float32),
        mesh=mesh,
        scratch_types=[pltpu.VMEM((_RCHUNK, n), jnp.float32)],
        compiler_params=pltpu.CompilerParams(needs_layout_passes=False),
    )(x2)
    return out.reshape(b, nh, n, n)


# speedup vs baseline: 4.7890x; 4.7768x over previous
import jax
import jax.numpy as jnp
from jax import lax
from jax.experimental import pallas as pl

_N = 577
_HB = 4

_RENORM_HEADS = (2, 3, 5, 7, 8)
_COPY_HEAD = 10


def _choose_attn_kernel(x_ref, o_ref):
    j = pl.program_id(1)
    x = x_ref[0]
    col = lax.broadcasted_iota(jnp.int32, x.shape, 2)
    t = jnp.where(col < _N, jnp.maximum(x, 0.0), 0.0)
    ones = jnp.ones((x.shape[2], 1), dtype=jnp.float32)
    s = lax.dot_general(
        t, ones, (((2,), (0,)), ((), ())), preferred_element_type=jnp.float32
    )
    renorm = t * (1.0 / (s + 1e-5))
    uniform = x * (1.0 / _N)

    hid = j * _HB + lax.broadcasted_iota(jnp.int32, (_HB, 1, 1), 0)
    is_renorm = hid == _RENORM_HEADS[0]
    for hh in _RENORM_HEADS[1:]:
        is_renorm = jnp.logical_or(is_renorm, hid == hh)
    is_copy = hid == _COPY_HEAD

    o_ref[0] = jnp.where(is_renorm, renorm, jnp.where(is_copy, x, uniform))


def kernel(attn_weights):
    b, nh, n, _ = attn_weights.shape
    grid = (b, nh // _HB)
    spec = pl.BlockSpec((1, _HB, n, n), lambda i, j: (i, j, 0, 0))
    return pl.pallas_call(
        _choose_attn_kernel,
        grid=grid,
        in_specs=[spec],
        out_specs=spec,
        out_shape=jax.ShapeDtypeStruct(attn_weights.shape, attn_weights.dtype),
    )(attn_weights)


# speedup vs baseline: 5.8590x; 1.2234x over previous
import jax
import jax.numpy as jnp
from jax import lax
from jax.experimental import pallas as pl

_N = 577
_HB = 4

_RENORM_HEADS = (2, 3, 5, 7, 8)
_COPY_HEAD = 10


def _choose_attn_kernel(x_ref, o_ref):
    j = pl.program_id(1)

    def write_group(g):
        for hi in range(_HB):
            head = g * _HB + hi
            x = x_ref[0, hi]
            if head in _RENORM_HEADS:
                col = lax.broadcasted_iota(jnp.int32, x.shape, 1)
                t = jnp.where(col < _N, jnp.maximum(x, 0.0), 0.0)
                s = jnp.sum(t, axis=1, keepdims=True)
                o_ref[0, hi] = t / (s + 1e-5)
            elif head == _COPY_HEAD:
                o_ref[0, hi] = x
            else:
                o_ref[0, hi] = x * (1.0 / _N)

    for g in range(12 // _HB):
        @pl.when(j == g)
        def _(g=g):
            write_group(g)


def kernel(attn_weights):
    b, nh, n, _ = attn_weights.shape
    grid = (b, nh // _HB)
    spec = pl.BlockSpec((1, _HB, n, n), lambda i, j: (i, j, 0, 0))
    return pl.pallas_call(
        _choose_attn_kernel,
        grid=grid,
        in_specs=[spec],
        out_specs=spec,
        out_shape=jax.ShapeDtypeStruct(attn_weights.shape, attn_weights.dtype),
    )(attn_weights)


# speedup vs baseline: 5.8655x; 1.0011x over previous
import jax
import jax.numpy as jnp
from jax import lax
from jax.experimental import pallas as pl

_N = 577
_HB = 6

_RENORM_HEADS = (2, 3, 5, 7, 8)
_COPY_HEAD = 10


def _choose_attn_kernel(x_ref, o_ref):
    j = pl.program_id(1)

    def write_group(g):
        for hi in range(_HB):
            head = g * _HB + hi
            x = x_ref[0, hi]
            if head in _RENORM_HEADS:
                col = lax.broadcasted_iota(jnp.int32, x.shape, 1)
                t = jnp.where(col < _N, jnp.maximum(x, 0.0), 0.0)
                s = jnp.sum(t, axis=1, keepdims=True)
                o_ref[0, hi] = t / (s + 1e-5)
            elif head == _COPY_HEAD:
                o_ref[0, hi] = x
            else:
                o_ref[0, hi] = x * (1.0 / _N)

    for g in range(12 // _HB):
        @pl.when(j == g)
        def _(g=g):
            write_group(g)


def kernel(attn_weights):
    b, nh, n, _ = attn_weights.shape
    grid = (b, nh // _HB)
    spec = pl.BlockSpec((1, _HB, n, n), lambda i, j: (i, j, 0, 0))
    return pl.pallas_call(
        _choose_attn_kernel,
        grid=grid,
        in_specs=[spec],
        out_specs=spec,
        out_shape=jax.ShapeDtypeStruct(attn_weights.shape, attn_weights.dtype),
    )(attn_weights)
